# baseline (speedup 1.0000x reference)
import math

import jax
import jax.numpy as jnp
from jax import lax
from jax.experimental import pallas as pl
from jax.experimental.pallas import tpu as pltpu
from jax.experimental.pallas import tpu_sc as plsc

_B, _N, _K, _A = 16, 2048, 8, 8
_R = _B * _N
_NC, _NS = 2, 16
_NTILES = _NC * _NS
_CH = 128


def _gather_sc(x, idx3):
    nchunk = idx3.shape[1]
    rpt = nchunk * _CH
    g = _NTILES * rpt
    mesh = plsc.VectorSubcoreMesh(core_axis_name="c", subcore_axis_name="s")

    nbuf = 4

    def body(x_hbm, idx_hbm, out_hbm, idx_v, *bufs):
        rows = bufs[:nbuf]
        gsems = bufs[nbuf:2 * nbuf]
        ssems = bufs[2 * nbuf:]
        wid = lax.axis_index("s") * _NC + lax.axis_index("c")
        pltpu.sync_copy(idx_hbm.at[wid], idx_v)

        def start(j, b):
            pltpu.async_copy(x_hbm.at[idx_v.at[j]], rows[b], gsems[b])

        def out_slice(j):
            return out_hbm.at[pl.ds(wid * rpt + j * _CH, _CH)]

        for b in range(nbuf - 1):
            start(b, b)

        def outer(jj, carry):
            j0 = jj * nbuf
            for b in range(nbuf):
                j = j0 + b
                pltpu.make_async_copy(
                    x_hbm.at[idx_v.at[j]], rows[b], gsems[b]).wait()
                pltpu.async_copy(rows[b], out_slice(j), ssems[b])
                jn = j + nbuf - 1

                @pl.when(jn < nchunk)
                def _():
                    bn = (b + nbuf - 1) % nbuf

                    @pl.when(jn >= nbuf)
                    def _():
                        pltpu.make_async_copy(
                            rows[bn], out_slice(jn - nbuf), ssems[bn]).wait()

                    start(jn, bn)
            return carry

        lax.fori_loop(0, nchunk // nbuf, outer, 0)
        for b in range(nbuf):
            pltpu.make_async_copy(
                rows[b], out_slice(nchunk - nbuf + b), ssems[b]).wait()

    return pl.kernel(
        body,
        out_type=jax.ShapeDtypeStruct((g, 128), jnp.float32),
        mesh=mesh,
        scratch_types=(
            [pltpu.VMEM((nchunk, 128), jnp.int32)]
            + [pltpu.VMEM((_CH, 128), jnp.float32) for _ in range(nbuf)]
            + [pltpu.SemaphoreType.DMA for _ in range(2 * nbuf)]
        ),
    )(x, idx3)


def _hop_rf(x, nbr, w, b, din):
    h = w.shape[1]
    br = 512

    def body(x_ref, nbr_ref, w_ref, b_ref, o_ref):
        ctr = x_ref[...][:, :din]
        nb = nbr_ref[...][:, :din]
        ctr_rep = jnp.broadcast_to(
            ctr[:, None, :], (br, _K, din)).reshape(br * _K, din)
        e = jnp.concatenate([ctr_rep, nb - ctr_rep], axis=1)
        hm = jnp.maximum(
            jnp.dot(e, w_ref[...], preferred_element_type=jnp.float32)
            + b_ref[...], 0.0)
        t = hm.reshape(br, _K, h)
        out = t[:, 0, :]
        for k in range(1, _K):
            out = jnp.maximum(out, t[:, k, :])
        if h < 128:
            out = jnp.concatenate(
                [out, jnp.zeros((br, 128 - h), jnp.float32)], axis=1)
        o_ref[...] = out

    rows = x.shape[0]
    return pl.pallas_call(
        body,
        grid=(rows // br,),
        in_specs=[
            pl.BlockSpec((br, 128), lambda i: (i, 0)),
            pl.BlockSpec((br * _K, 128), lambda i: (i, 0)),
            pl.BlockSpec((2 * din, h), lambda i: (0, 0)),
            pl.BlockSpec((1, h), lambda i: (0, 0)),
        ],
        out_specs=pl.BlockSpec((br, 128), lambda i: (i, 0)),
        out_shape=jax.ShapeDtypeStruct((rows, 128), jnp.float32),
    )(x, nbr, w, b)


def _final(x, wg, bg, qt, gum):
    h = wg.shape[1]

    def body(x_ref, wg_ref, bg_ref, qt_ref, g_ref, probs_ref, samp_ref):
        xv = x_ref[...]
        gmean = jnp.mean(xv, axis=0, keepdims=True)
        e2 = jnp.concatenate(
            [xv, jnp.broadcast_to(gmean, (_N, h))], axis=1)
        xg = jnp.maximum(
            jnp.dot(e2, wg_ref[...], preferred_element_type=jnp.float32)
            + bg_ref[...], 0.0)
        s = jnp.dot(xg, qt_ref[...],
                    preferred_element_type=jnp.float32) / math.sqrt(float(h))
        mx = jnp.max(s, axis=-1, keepdims=True)
        e = jnp.exp(s - mx)
        probs = e / jnp.sum(e, axis=-1, keepdims=True)
        probs_ref[...] = probs
        y = jnp.log(probs + 1e-9) + g_ref[...]
        my = jnp.max(y, axis=-1, keepdims=True)
        ii = lax.broadcasted_iota(jnp.int32, y.shape, 1)
        cand = jnp.where(y >= my, ii, _A)
        samp_ref[...] = jnp.min(cand, axis=-1, keepdims=True)

    rows = x.shape[0]
    return pl.pallas_call(
        body,
        grid=(rows // _N,),
        in_specs=[
            pl.BlockSpec((_N, h), lambda i: (i, 0)),
            pl.BlockSpec((2 * h, h), lambda i: (0, 0)),
            pl.BlockSpec((1, h), lambda i: (0, 0)),
            pl.BlockSpec((h, _A), lambda i: (0, 0)),
            pl.BlockSpec((_N, _A), lambda i: (i, 0)),
        ],
        out_specs=[
            pl.BlockSpec((_N, _A), lambda i: (i, 0)),
            pl.BlockSpec((_N, 1), lambda i: (i, 0)),
        ],
        out_shape=[
            jax.ShapeDtypeStruct((rows, _A), jnp.float32),
            jax.ShapeDtypeStruct((rows, 1), jnp.int32),
        ],
    )(x, wg, bg, qt, gum)


def _half_chain(nf_half, gidx3, weights, gum_half):
    W1, b1, W2, b2, W3, b3, Wc, bc, Wg, bg, qt = weights
    x = jnp.pad(nf_half, ((0, 0), (0, 126)))
    x = _hop_rf(x, _gather_sc(x, gidx3), W1, b1, 2)
    x = _hop_rf(x, _gather_sc(x, gidx3), W2, b2, 64)
    x = _hop_rf(x, _gather_sc(x, gidx3), W3, b3, 128)
    x = _hop_rf(x, _gather_sc(x, gidx3), Wc, bc, 128)
    return _final(x, Wg, bg, qt, gum_half)


def kernel(nfeature, nn_idx, W1, b1, W2, b2, W3, b3, Wc, bc, Wg, bg, Q):
    hb = _B // 2
    hr = hb * _N
    loc = (jnp.arange(hb, dtype=jnp.int32) * _N)[:, None, None]
    gidx_a = (nn_idx[:hb].astype(jnp.int32) + loc).reshape(_NTILES, -1, 128)
    gidx_b = (nn_idx[hb:].astype(jnp.int32) + loc).reshape(_NTILES, -1, 128)
    gum = jax.random.gumbel(jax.random.key(42), (_R, _A), jnp.float32)
    nf2 = nfeature.reshape(_R, 2)

    weights = (W1, b1.reshape(1, -1), W2, b2.reshape(1, -1),
               W3, b3.reshape(1, -1), Wc, bc.reshape(1, -1),
               Wg, bg.reshape(1, -1), Q.T)
    pa, sa = _half_chain(nf2[:hr], gidx_a, weights, gum[:hr])
    pb, sb = _half_chain(nf2[hr:], gidx_b, weights, gum[hr:])
    probs = jnp.concatenate([pa, pb], axis=0).reshape(_B, _N, _A)
    samp = jnp.concatenate([sa, sb], axis=0).reshape(_B, 1, _N)
    return probs, samp


# speedup vs baseline: 1.0596x; 1.0596x over previous
import math

import jax
import jax.numpy as jnp
from jax import lax
from jax.experimental import pallas as pl
from jax.experimental.pallas import tpu as pltpu
from jax.experimental.pallas import tpu_sc as plsc

_B, _N, _K, _A = 16, 2048, 8, 8
_R = _B * _N
_NC, _NS = 2, 16
_NTILES = _NC * _NS
_CH = 128


def _gather_sc(x, idx3):
    nchunk = idx3.shape[1]
    rpt = nchunk * _CH
    g = _NTILES * rpt
    mesh = plsc.VectorSubcoreMesh(core_axis_name="c", subcore_axis_name="s")

    def body(x_hbm, idx_hbm, out_hbm, idx_v, rows0, rows1, sem0, sem1):
        wid = lax.axis_index("s") * _NC + lax.axis_index("c")
        pltpu.sync_copy(idx_hbm.at[wid], idx_v)
        rows = (rows0, rows1)
        sems = (sem0, sem1)

        def start(j, buf):
            pltpu.async_copy(x_hbm.at[idx_v.at[j]], rows[buf], sems[buf])

        def finish(j, buf):
            pltpu.make_async_copy(
                x_hbm.at[idx_v.at[j]], rows[buf], sems[buf]).wait()
            pltpu.sync_copy(
                rows[buf], out_hbm.at[pl.ds(wid * rpt + j * _CH, _CH)])

        start(0, 0)

        def outer(jj, carry):
            j0 = jj * 2
            start(j0 + 1, 1)
            finish(j0, 0)

            @pl.when(j0 + 2 < nchunk)
            def _():
                start(j0 + 2, 0)

            finish(j0 + 1, 1)
            return carry

        lax.fori_loop(0, nchunk // 2, outer, 0)

    return pl.kernel(
        body,
        out_type=jax.ShapeDtypeStruct((g, 128), jnp.float32),
        mesh=mesh,
        scratch_types=[
            pltpu.VMEM((nchunk, 128), jnp.int32),
            pltpu.VMEM((_CH, 128), jnp.float32),
            pltpu.VMEM((_CH, 128), jnp.float32),
            pltpu.SemaphoreType.DMA,
            pltpu.SemaphoreType.DMA,
        ],
    )(x, idx3)


def _hop_rf(x, nbr, w, b, din):
    h = w.shape[1]
    br = 1024

    def body(x_ref, nbr_ref, w_ref, b_ref, o_ref):
        ctr = x_ref[...][:, :din]
        nb = nbr_ref[...][:, :din]
        ctr_rep = jnp.broadcast_to(
            ctr[:, None, :], (br, _K, din)).reshape(br * _K, din)
        e = jnp.concatenate([ctr_rep, nb - ctr_rep], axis=1)
        hm = jnp.maximum(
            jnp.dot(e, w_ref[...], preferred_element_type=jnp.float32)
            + b_ref[...], 0.0)
        t = hm.reshape(br, _K, h)
        out = t[:, 0, :]
        for k in range(1, _K):
            out = jnp.maximum(out, t[:, k, :])
        if h < 128:
            out = jnp.concatenate(
                [out, jnp.zeros((br, 128 - h), jnp.float32)], axis=1)
        o_ref[...] = out

    rows = x.shape[0]
    return pl.pallas_call(
        body,
        grid=(rows // br,),
        in_specs=[
            pl.BlockSpec((br, 128), lambda i: (i, 0)),
            pl.BlockSpec((br * _K, 128), lambda i: (i, 0)),
            pl.BlockSpec((2 * din, h), lambda i: (0, 0)),
            pl.BlockSpec((1, h), lambda i: (0, 0)),
        ],
        out_specs=pl.BlockSpec((br, 128), lambda i: (i, 0)),
        out_shape=jax.ShapeDtypeStruct((rows, 128), jnp.float32),
    )(x, nbr, w, b)


def _final(x, wg, bg, qt, gum):
    h = wg.shape[1]

    def body(x_ref, wg_ref, bg_ref, qt_ref, g_ref, probs_ref, samp_ref):
        xv = x_ref[...]
        gmean = jnp.mean(xv, axis=0, keepdims=True)
        e2 = jnp.concatenate(
            [xv, jnp.broadcast_to(gmean, (_N, h))], axis=1)
        xg = jnp.maximum(
            jnp.dot(e2, wg_ref[...], preferred_element_type=jnp.float32)
            + bg_ref[...], 0.0)
        s = jnp.dot(xg, qt_ref[...],
                    preferred_element_type=jnp.float32) / math.sqrt(float(h))
        mx = jnp.max(s, axis=-1, keepdims=True)
        e = jnp.exp(s - mx)
        probs = e / jnp.sum(e, axis=-1, keepdims=True)
        probs_ref[...] = probs
        y = jnp.log(probs + 1e-9) + g_ref[...]
        my = jnp.max(y, axis=-1, keepdims=True)
        ii = lax.broadcasted_iota(jnp.int32, y.shape, 1)
        cand = jnp.where(y >= my, ii, _A)
        samp_ref[...] = jnp.min(cand, axis=-1, keepdims=True)

    rows = x.shape[0]
    return pl.pallas_call(
        body,
        grid=(rows // _N,),
        in_specs=[
            pl.BlockSpec((_N, h), lambda i: (i, 0)),
            pl.BlockSpec((2 * h, h), lambda i: (0, 0)),
            pl.BlockSpec((1, h), lambda i: (0, 0)),
            pl.BlockSpec((h, _A), lambda i: (0, 0)),
            pl.BlockSpec((_N, _A), lambda i: (i, 0)),
        ],
        out_specs=[
            pl.BlockSpec((_N, _A), lambda i: (i, 0)),
            pl.BlockSpec((_N, 1), lambda i: (i, 0)),
        ],
        out_shape=[
            jax.ShapeDtypeStruct((rows, _A), jnp.float32),
            jax.ShapeDtypeStruct((rows, 1), jnp.int32),
        ],
    )(x, wg, bg, qt, gum)


def _half_chain(nf_half, gidx3, weights, gum_half):
    W1, b1, W2, b2, W3, b3, Wc, bc, Wg, bg, qt = weights
    x = jnp.pad(nf_half, ((0, 0), (0, 126)))
    x = _hop_rf(x, _gather_sc(x, gidx3), W1, b1, 2)
    x = _hop_rf(x, _gather_sc(x, gidx3), W2, b2, 64)
    x = _hop_rf(x, _gather_sc(x, gidx3), W3, b3, 128)
    x = _hop_rf(x, _gather_sc(x, gidx3), Wc, bc, 128)
    return _final(x, Wg, bg, qt, gum_half)


def kernel(nfeature, nn_idx, W1, b1, W2, b2, W3, b3, Wc, bc, Wg, bg, Q):
    hb = _B // 2
    hr = hb * _N
    loc = (jnp.arange(hb, dtype=jnp.int32) * _N)[:, None, None]
    gidx_a = (nn_idx[:hb].astype(jnp.int32) + loc).reshape(_NTILES, -1, 128)
    gidx_b = (nn_idx[hb:].astype(jnp.int32) + loc).reshape(_NTILES, -1, 128)
    gum = jax.random.gumbel(jax.random.key(42), (_R, _A), jnp.float32)
    nf2 = nfeature.reshape(_R, 2)

    weights = (W1, b1.reshape(1, -1), W2, b2.reshape(1, -1),
               W3, b3.reshape(1, -1), Wc, bc.reshape(1, -1),
               Wg, bg.reshape(1, -1), Q.T)
    pa, sa = _half_chain(nf2[:hr], gidx_a, weights, gum[:hr])
    pb, sb = _half_chain(nf2[hr:], gidx_b, weights, gum[hr:])
    probs = jnp.concatenate([pa, pb], axis=0).reshape(_B, _N, _A)
    samp = jnp.concatenate([sa, sb], axis=0).reshape(_B, 1, _N)
    return probs, samp


# speedup vs baseline: 1.0718x; 1.0115x over previous
import math

import jax
import jax.numpy as jnp
from jax import lax
from jax.experimental import pallas as pl
from jax.experimental.pallas import tpu as pltpu
from jax.experimental.pallas import tpu_sc as plsc

_B, _N, _K, _A = 16, 2048, 8, 8
_R = _B * _N
_NC, _NS = 2, 16
_NTILES = _NC * _NS
_CH = 128


def _gather_sc(x, idx3):
    nchunk = idx3.shape[1]
    rpt = nchunk * _CH
    g = _NTILES * rpt
    mesh = plsc.VectorSubcoreMesh(core_axis_name="c", subcore_axis_name="s")

    def body(x_hbm, idx_hbm, out_hbm, idx_v, rows0, rows1, sem0, sem1):
        wid = lax.axis_index("s") * _NC + lax.axis_index("c")
        pltpu.sync_copy(idx_hbm.at[wid], idx_v)
        rows = (rows0, rows1)
        sems = (sem0, sem1)

        def start(j, buf):
            pltpu.async_copy(x_hbm.at[idx_v.at[j]], rows[buf], sems[buf])

        def finish(j, buf):
            pltpu.make_async_copy(
                x_hbm.at[idx_v.at[j]], rows[buf], sems[buf]).wait()
            pltpu.sync_copy(
                rows[buf], out_hbm.at[pl.ds(wid * rpt + j * _CH, _CH)])

        start(0, 0)

        def outer(jj, carry):
            j0 = jj * 2
            start(j0 + 1, 1)
            finish(j0, 0)

            @pl.when(j0 + 2 < nchunk)
            def _():
                start(j0 + 2, 0)

            finish(j0 + 1, 1)
            return carry

        lax.fori_loop(0, nchunk // 2, outer, 0)

    return pl.kernel(
        body,
        out_type=jax.ShapeDtypeStruct((g, 128), jnp.float32),
        mesh=mesh,
        scratch_types=[
            pltpu.VMEM((nchunk, 128), jnp.int32),
            pltpu.VMEM((_CH, 128), jnp.float32),
            pltpu.VMEM((_CH, 128), jnp.float32),
            pltpu.SemaphoreType.DMA,
            pltpu.SemaphoreType.DMA,
        ],
    )(x, idx3)


def _hop_rf(x, nbr, w, b, din):
    h = w.shape[1]
    br = 2048

    def body(x_ref, nbr_ref, w_ref, b_ref, o_ref):
        ctr = x_ref[...][:, :din]
        nb = nbr_ref[...][:, :din]
        ctr_rep = jnp.broadcast_to(
            ctr[:, None, :], (br, _K, din)).reshape(br * _K, din)
        e = jnp.concatenate([ctr_rep, nb - ctr_rep], axis=1)
        hm = jnp.maximum(
            jnp.dot(e, w_ref[...], preferred_element_type=jnp.float32)
            + b_ref[...], 0.0)
        t = hm.reshape(br, _K, h)
        out = t[:, 0, :]
        for k in range(1, _K):
            out = jnp.maximum(out, t[:, k, :])
        if h < 128:
            out = jnp.concatenate(
                [out, jnp.zeros((br, 128 - h), jnp.float32)], axis=1)
        o_ref[...] = out

    rows = x.shape[0]
    return pl.pallas_call(
        body,
        grid=(rows // br,),
        in_specs=[
            pl.BlockSpec((br, 128), lambda i: (i, 0)),
            pl.BlockSpec((br * _K, 128), lambda i: (i, 0)),
            pl.BlockSpec((2 * din, h), lambda i: (0, 0)),
            pl.BlockSpec((1, h), lambda i: (0, 0)),
        ],
        out_specs=pl.BlockSpec((br, 128), lambda i: (i, 0)),
        out_shape=jax.ShapeDtypeStruct((rows, 128), jnp.float32),
    )(x, nbr, w, b)


def _final(x, wg, bg, qt, gum):
    h = wg.shape[1]

    def body(x_ref, wg_ref, bg_ref, qt_ref, g_ref, probs_ref, samp_ref):
        xv = x_ref[...]
        gmean = jnp.mean(xv, axis=0, keepdims=True)
        e2 = jnp.concatenate(
            [xv, jnp.broadcast_to(gmean, (_N, h))], axis=1)
        xg = jnp.maximum(
            jnp.dot(e2, wg_ref[...], preferred_element_type=jnp.float32)
            + bg_ref[...], 0.0)
        s = jnp.dot(xg, qt_ref[...],
                    preferred_element_type=jnp.float32) / math.sqrt(float(h))
        mx = jnp.max(s, axis=-1, keepdims=True)
        e = jnp.exp(s - mx)
        probs = e / jnp.sum(e, axis=-1, keepdims=True)
        probs_ref[...] = probs
        y = jnp.log(probs + 1e-9) + g_ref[...]
        my = jnp.max(y, axis=-1, keepdims=True)
        ii = lax.broadcasted_iota(jnp.int32, y.shape, 1)
        cand = jnp.where(y >= my, ii, _A)
        samp_ref[...] = jnp.min(cand, axis=-1, keepdims=True)

    rows = x.shape[0]
    return pl.pallas_call(
        body,
        grid=(rows // _N,),
        in_specs=[
            pl.BlockSpec((_N, h), lambda i: (i, 0)),
            pl.BlockSpec((2 * h, h), lambda i: (0, 0)),
            pl.BlockSpec((1, h), lambda i: (0, 0)),
            pl.BlockSpec((h, _A), lambda i: (0, 0)),
            pl.BlockSpec((_N, _A), lambda i: (i, 0)),
        ],
        out_specs=[
            pl.BlockSpec((_N, _A), lambda i: (i, 0)),
            pl.BlockSpec((_N, 1), lambda i: (i, 0)),
        ],
        out_shape=[
            jax.ShapeDtypeStruct((rows, _A), jnp.float32),
            jax.ShapeDtypeStruct((rows, 1), jnp.int32),
        ],
    )(x, wg, bg, qt, gum)


def _half_chain(nf_half, gidx3, weights, gum_half):
    W1, b1, W2, b2, W3, b3, Wc, bc, Wg, bg, qt = weights
    x = jnp.pad(nf_half, ((0, 0), (0, 126)))
    x = _hop_rf(x, _gather_sc(x, gidx3), W1, b1, 2)
    x = _hop_rf(x, _gather_sc(x, gidx3), W2, b2, 64)
    x = _hop_rf(x, _gather_sc(x, gidx3), W3, b3, 128)
    x = _hop_rf(x, _gather_sc(x, gidx3), Wc, bc, 128)
    return _final(x, Wg, bg, qt, gum_half)


def kernel(nfeature, nn_idx, W1, b1, W2, b2, W3, b3, Wc, bc, Wg, bg, Q):
    hb = _B // 2
    hr = hb * _N
    loc = (jnp.arange(hb, dtype=jnp.int32) * _N)[:, None, None]
    gidx_a = (nn_idx[:hb].astype(jnp.int32) + loc).reshape(_NTILES, -1, 128)
    gidx_b = (nn_idx[hb:].astype(jnp.int32) + loc).reshape(_NTILES, -1, 128)
    gum = jax.random.gumbel(jax.random.key(42), (_R, _A), jnp.float32)
    nf2 = nfeature.reshape(_R, 2)

    weights = (W1, b1.reshape(1, -1), W2, b2.reshape(1, -1),
               W3, b3.reshape(1, -1), Wc, bc.reshape(1, -1),
               Wg, bg.reshape(1, -1), Q.T)
    pa, sa = _half_chain(nf2[:hr], gidx_a, weights, gum[:hr])
    pb, sb = _half_chain(nf2[hr:], gidx_b, weights, gum[hr:])
    probs = jnp.concatenate([pa, pb], axis=0).reshape(_B, _N, _A)
    samp = jnp.concatenate([sa, sb], axis=0).reshape(_B, 1, _N)
    return probs, samp
